# baseline (speedup 1.0000x reference)
import functools

import jax
import jax.numpy as jnp
from jax import lax
from jax.experimental import pallas as pl
from jax.experimental.pallas import tpu as pltpu
from jax.experimental.pallas import tpu_sc as plsc

N = 50000
NP = 50048
E = 800000
EP = 819200
EMB = 64
NTILES = 16
CHUNK = 1024
EPT = EP // NTILES
NCHUNK = EPT // CHUNK
ROWS_PT = NP // NTILES
BLK = 3128


def _lane_splat(vec, l):
    return lax.gather(
        vec, jnp.full((16, 1), l, jnp.int32),
        lax.GatherDimensionNumbers(
            offset_dims=(), collapsed_slice_dims=(0,), start_index_map=(0,)),
        slice_sizes=(1,),
        mode=lax.GatherScatterMode.PROMISE_IN_BOUNDS)


def _make_sc_edge():
    mesh = plsc.VectorSubcoreMesh(core_axis_name="c", subcore_axis_name="s")

    @functools.partial(
        pl.kernel,
        mesh=mesh,
        compiler_params=pltpu.CompilerParams(use_tc_tiling_on_sc=False),
        out_type=[
            jax.ShapeDtypeStruct((4 * NP, 16), jnp.float32),
            jax.ShapeDtypeStruct((NP, 1), jnp.float32),
        ],
        scratch_types=[
            pltpu.VMEM((8, 128), jnp.int32),
            pltpu.VMEM((8, 128), jnp.int32),
            pltpu.VMEM((8, 128), jnp.int32),
            pltpu.VMEM((CHUNK // 16, 16), jnp.float32),
            pltpu.VMEM((CHUNK, 16), jnp.float32),
            pltpu.VMEM((CHUNK, 16), jnp.float32),
            pltpu.VMEM((CHUNK, 16), jnp.float32),
            pltpu.VMEM((16,), jnp.float32),
            pltpu.VMEM((128, 1), jnp.float32),
            pltpu.VMEM_SHARED((NP, 16), jnp.float32),
            pltpu.VMEM_SHARED((NP, 1), jnp.float32),
            pltpu.SemaphoreType.DMA,
        ],
    )
    def sc_edge(dst_h, src_h, e_h, tab_a, tab_b, fe_h, z_s, z_d,
                ones_h, s_out, deg_out,
                dst_v, dsto_v, srco_v, e_v, a_v, b_v, t_v, fe_v, ones_v,
                s_sh, deg_sh, sem):
        cid = lax.axis_index("c")
        sid = lax.axis_index("s")
        r0 = sid * ROWS_PT
        pltpu.sync_copy(ones_h, ones_v)

        @pl.when(cid == 0)
        def _():
            pltpu.sync_copy(z_d.at[pl.ds(r0, ROWS_PT)],
                            deg_sh.at[pl.ds(r0, ROWS_PT)])

        for half in (0, 1):
            q = 2 * half + cid
            qoff = q * NP

            pltpu.sync_copy(z_s.at[pl.ds(r0, ROWS_PT)],
                            s_sh.at[pl.ds(r0, ROWS_PT)])
            pltpu.sync_copy(fe_h.at[pl.ds(q * 16, 16)], fe_v)
            plsc.subcore_barrier()

            def chunk_body(ck, carry):
                erow = sid * (EPT // 128) + ck * 8
                erow16 = sid * (EPT // 16) + ck * (CHUNK // 16)
                pltpu.sync_copy(dst_h.at[pl.ds(erow, 8)], dst_v)
                pltpu.sync_copy(src_h.at[pl.ds(erow, 8)], srco_v)
                pltpu.sync_copy(e_h.at[pl.ds(erow16, CHUNK // 16)], e_v)

                def off_body(r, c2):
                    for k in range(8):
                        sl = pl.ds(k * 16, 16)
                        dsto_v[r, sl] = dst_v[r, sl] + qoff
                        srco_v[r, sl] = srco_v[r, sl] + qoff
                    return c2

                lax.fori_loop(0, 8, off_body, 0)

                copies = []
                for j in range(8):
                    copies.append(pltpu.async_copy(
                        tab_a.at[dsto_v.at[j]],
                        a_v.at[pl.ds(j * 128, 128)], sem))
                    copies.append(pltpu.async_copy(
                        tab_b.at[srco_v.at[j]],
                        b_v.at[pl.ds(j * 128, 128)], sem))
                for cp in copies:
                    cp.wait()

                fe16 = fe_v[...]

                @plsc.parallel_loop(0, CHUNK // 16, unroll=2)
                def _(g):
                    ev16 = e_v[g]
                    for l in range(16):
                        i = g * 16 + l
                        x2 = a_v[i] + b_v[i] + _lane_splat(ev16, l) * fe16
                        t_v[i] = 1.0 - 2.0 / (jnp.exp(x2) + 1.0)

                for j in range(8):
                    pltpu.sync_copy(t_v.at[pl.ds(j * 128, 128)],
                                    s_sh.at[dst_v.at[j]], add=True)

                if half == 0:
                    @pl.when(cid == 0)
                    def _():
                        for j in range(8):
                            pltpu.sync_copy(ones_v, deg_sh.at[dst_v.at[j]],
                                            add=True)

                return carry

            lax.fori_loop(0, NCHUNK, chunk_body, 0)
            plsc.subcore_barrier()

            pltpu.sync_copy(s_sh.at[pl.ds(r0, ROWS_PT)],
                            s_out.at[pl.ds(qoff + r0, ROWS_PT)])
            if half == 0:
                plsc.subcore_barrier()

        @pl.when(cid == 0)
        def _():
            pltpu.sync_copy(deg_sh.at[pl.ds(r0, ROWS_PT)],
                            deg_out.at[pl.ds(r0, ROWS_PT)])

    return sc_edge


_sc_edge = _make_sc_edge()


def _dot(a, b):
    return jnp.dot(a, b, preferred_element_type=jnp.float32)


def _quarters(out_ref, x):
    for qq in range(4):
        out_ref[qq] = x[:, qq * 16:(qq + 1) * 16]


def _tc1_body(cf_ref, vf_ref, cw1, cb1, cw2, cb2, vw1, vb1, vw2, vb2,
              a1w, a1b, b1w, a2w, a2b,
              c_out, v_out, a1_out, b1_out, a2_out, zs_out, zd_out):
    zs_out[...] = jnp.zeros_like(zs_out)
    zd_out[...] = jnp.zeros_like(zd_out)
    c = jnp.tanh(_dot(jnp.tanh(_dot(cf_ref[...], cw1[...]) + cb1[...]),
                      cw2[...]) + cb2[...])
    v = jnp.tanh(_dot(jnp.tanh(_dot(vf_ref[...], vw1[...]) + vb1[...]),
                      vw2[...]) + vb2[...])
    c_out[...] = c
    v_out[...] = v
    _quarters(a1_out, 2.0 * (_dot(c, a1w[...]) + a1b[...]))
    _quarters(b1_out, 2.0 * _dot(v, b1w[...]))
    _quarters(a2_out, 2.0 * (_dot(v, a2w[...]) + a2b[...]))


def _tc2_body(s_ref, deg_ref, c_ref, ffo1, fbo1, o1bot, o1b, o2w, o2b, frw,
              b2_out):
    s = jnp.concatenate([s_ref[0], s_ref[1], s_ref[2], s_ref[3]], axis=-1)
    z = (_dot(s, ffo1[...]) + deg_ref[...] * fbo1[...]
         + _dot(c_ref[...], o1bot[...]) + o1b[...])
    cnew = _dot(jnp.tanh(z), o2w[...]) + o2b[...]
    _quarters(b2_out, 2.0 * _dot(cnew, frw[...]))


def _tc3_body(s_ref, deg_ref, v_ref, ffo1, fbo1, o1bot, o1b, o2w, o2b,
              w1, b1, w2, out_ref):
    s = jnp.concatenate([s_ref[0], s_ref[1], s_ref[2], s_ref[3]], axis=-1)
    z = (_dot(s, ffo1[...]) + deg_ref[...] * fbo1[...]
         + _dot(v_ref[...], o1bot[...]) + o1b[...])
    vnew = _dot(jnp.tanh(z), o2w[...]) + o2b[...]
    out_ref[...] = _dot(jnp.tanh(_dot(vnew, w1[...]) + b1[...]), w2[...])


def _row_spec(width):
    return pl.BlockSpec((BLK, width), lambda i: (i, 0))


def _q_spec():
    return pl.BlockSpec((4, BLK, 16), lambda i: (0, i, 0))


def _w_spec(shape):
    nd = len(shape)
    return pl.BlockSpec(shape, lambda i, _n=nd: (0,) * _n)


def kernel(constraint_features, edge_indices, edge_features,
           variable_features, params):
    p = params
    f32 = jnp.float32

    cfp = jnp.pad(constraint_features, ((0, NP - N), (0, 3)))
    vfp = jnp.pad(variable_features, ((0, NP - N), (0, 7)))
    cw1 = jnp.pad(p["c_emb"]["W1"], ((0, 3), (0, 0)))
    vw1 = jnp.pad(p["v_emb"]["W1"], ((0, 7), (0, 0)))

    def row(x):
        return x.reshape(1, -1).astype(f32)

    pad_i = jnp.full((EP - E,), N, jnp.int32)
    d1 = jnp.concatenate([edge_indices[0], pad_i])
    s1 = jnp.concatenate([edge_indices[1], pad_i])
    dst1 = d1.reshape(EP // 128, 128)
    dst2 = s1.reshape(EP // 128, 128)
    e_pad = jnp.concatenate([edge_features[:, 0],
                             jnp.zeros((EP - E,), f32)]).reshape(EP // 16, 16)
    ones_h = jnp.ones((128, 1), f32)
    fe1 = 2.0 * p["vc"]["fe_W"].reshape(EMB)
    fe2 = 2.0 * p["cv"]["fe_W"].reshape(EMB)

    def conv_consts(q):
        o1t, o1bot = q["o1_W"][:EMB], q["o1_W"][EMB:]
        return (q["ff_W"] @ o1t, row(q["ff_b"] @ o1t), o1bot,
                row(q["o1_b"]), q["o2_W"], row(q["o2_b"]))

    ffo1_1, fbo1_1, o1bot_1, o1b_1, o2w_1, o2b_1 = conv_consts(p["vc"])
    ffo1_2, fbo1_2, o1bot_2, o1b_2, o2w_2, o2b_2 = conv_consts(p["cv"])

    grid = NP // BLK
    w = _w_spec
    c, v, a1, b1, a2, z_s, z_d = pl.pallas_call(
        _tc1_body,
        grid=(grid,),
        in_specs=[_row_spec(8), _row_spec(24),
                  w((8, EMB)), w((1, EMB)), w((EMB, EMB)), w((1, EMB)),
                  w((24, EMB)), w((1, EMB)), w((EMB, EMB)), w((1, EMB)),
                  w((EMB, EMB)), w((1, EMB)), w((EMB, EMB)),
                  w((EMB, EMB)), w((1, EMB))],
        out_specs=[_row_spec(EMB), _row_spec(EMB),
                   _q_spec(), _q_spec(), _q_spec(),
                   _row_spec(16), _row_spec(1)],
        out_shape=[jax.ShapeDtypeStruct((NP, EMB), f32),
                   jax.ShapeDtypeStruct((NP, EMB), f32),
                   jax.ShapeDtypeStruct((4, NP, 16), f32),
                   jax.ShapeDtypeStruct((4, NP, 16), f32),
                   jax.ShapeDtypeStruct((4, NP, 16), f32),
                   jax.ShapeDtypeStruct((NP, 16), f32),
                   jax.ShapeDtypeStruct((NP, 1), f32)],
    )(cfp, vfp,
      cw1, row(p["c_emb"]["b1"]), p["c_emb"]["W2"], row(p["c_emb"]["b2"]),
      vw1, row(p["v_emb"]["b1"]), p["v_emb"]["W2"], row(p["v_emb"]["b2"]),
      p["vc"]["fl_W"], row(p["vc"]["fl_b"]), p["vc"]["fr_W"],
      p["cv"]["fl_W"], row(p["cv"]["fl_b"]))

    s1_out, deg1 = _sc_edge(dst1, dst2, e_pad,
                            a1.reshape(4 * NP, 16), b1.reshape(4 * NP, 16),
                            fe1, z_s, z_d, ones_h)

    (b2,) = pl.pallas_call(
        _tc2_body,
        grid=(grid,),
        in_specs=[_q_spec(), _row_spec(1), _row_spec(EMB),
                  w((EMB, EMB)), w((1, EMB)), w((EMB, EMB)), w((1, EMB)),
                  w((EMB, EMB)), w((1, EMB)), w((EMB, EMB))],
        out_specs=[_q_spec()],
        out_shape=[jax.ShapeDtypeStruct((4, NP, 16), f32)],
    )(s1_out.reshape(4, NP, 16), deg1, c,
      ffo1_1, fbo1_1, o1bot_1, o1b_1, o2w_1, o2b_1, p["cv"]["fr_W"])

    s2_out, deg2 = _sc_edge(dst2, dst1, e_pad,
                            a2.reshape(4 * NP, 16), b2.reshape(4 * NP, 16),
                            fe2, z_s, z_d, ones_h)

    (out,) = pl.pallas_call(
        _tc3_body,
        grid=(grid,),
        in_specs=[_q_spec(), _row_spec(1), _row_spec(EMB),
                  w((EMB, EMB)), w((1, EMB)), w((EMB, EMB)), w((1, EMB)),
                  w((EMB, EMB)), w((1, EMB)),
                  w((EMB, EMB)), w((1, EMB)), w((EMB, 1))],
        out_specs=[_row_spec(1)],
        out_shape=[jax.ShapeDtypeStruct((NP, 1), f32)],
    )(s2_out.reshape(4, NP, 16), deg2, v,
      ffo1_2, fbo1_2, o1bot_2, o1b_2, o2w_2, o2b_2,
      p["out"]["W1"], row(p["out"]["b1"]), p["out"]["W2"])

    return out[:N]


# speedup vs baseline: 1.1631x; 1.1631x over previous
import functools

import jax
import jax.numpy as jnp
from jax import lax
from jax.experimental import pallas as pl
from jax.experimental.pallas import tpu as pltpu
from jax.experimental.pallas import tpu_sc as plsc

N = 50000
NP = 50048
E = 800000
EP = 819200
EMB = 64
NTILES = 16
CHUNK = 1024
EPT = EP // NTILES
NCHUNK = EPT // CHUNK
ROWS_PT = NP // NTILES
BLK = 3128


def _lane_splat(vec, l):
    return lax.gather(
        vec, jnp.full((16, 1), l, jnp.int32),
        lax.GatherDimensionNumbers(
            offset_dims=(), collapsed_slice_dims=(0,), start_index_map=(0,)),
        slice_sizes=(1,),
        mode=lax.GatherScatterMode.PROMISE_IN_BOUNDS)


def _make_sc_edge():
    mesh = plsc.VectorSubcoreMesh(core_axis_name="c", subcore_axis_name="s")

    @functools.partial(
        pl.kernel,
        mesh=mesh,
        compiler_params=pltpu.CompilerParams(use_tc_tiling_on_sc=False),
        out_type=[
            jax.ShapeDtypeStruct((4 * NP, 16), jnp.float32),
            jax.ShapeDtypeStruct((NP, 1), jnp.float32),
        ],
        scratch_types=[
            pltpu.VMEM((8, 128), jnp.int32),
            pltpu.VMEM((8, 128), jnp.int32),
            pltpu.VMEM((8, 128), jnp.int32),
            pltpu.VMEM((CHUNK // 16, 16), jnp.float32),
            pltpu.VMEM((CHUNK, 16), jnp.float32),
            pltpu.VMEM((CHUNK, 16), jnp.float32),
            pltpu.VMEM((CHUNK, 16), jnp.float32),
            pltpu.VMEM((16,), jnp.float32),
            pltpu.VMEM((128, 1), jnp.float32),
            pltpu.VMEM_SHARED((NP, 16), jnp.float32),
            pltpu.VMEM_SHARED((NP, 1), jnp.float32),
            pltpu.SemaphoreType.DMA,
        ],
    )
    def sc_edge(dst_h, src_h, e_h, tab_a, tab_b, fe_h, z_s, z_d,
                ones_h, s_out, deg_out,
                dst_v, dsto_v, srco_v, e_v, a_v, b_v, t_v, fe_v, ones_v,
                s_sh, deg_sh, sem):
        cid = lax.axis_index("c")
        sid = lax.axis_index("s")
        r0 = sid * ROWS_PT
        pltpu.sync_copy(ones_h, ones_v)

        @pl.when(cid == 0)
        def _():
            pltpu.sync_copy(z_d.at[pl.ds(r0, ROWS_PT)],
                            deg_sh.at[pl.ds(r0, ROWS_PT)])

        for half in (0, 1):
            q = 2 * half + cid
            qoff = q * NP

            pltpu.sync_copy(z_s.at[pl.ds(r0, ROWS_PT)],
                            s_sh.at[pl.ds(r0, ROWS_PT)])
            pltpu.sync_copy(fe_h.at[pl.ds(q * 16, 16)], fe_v)
            plsc.subcore_barrier()

            def chunk_body(ck, carry):
                erow = sid * (EPT // 128) + ck * 8
                erow16 = sid * (EPT // 16) + ck * (CHUNK // 16)
                i1 = pltpu.async_copy(dst_h.at[pl.ds(erow, 8)], dst_v,
                                      sem)
                i2 = pltpu.async_copy(src_h.at[pl.ds(erow, 8)], srco_v,
                                      sem)
                i3 = pltpu.async_copy(e_h.at[pl.ds(erow16, CHUNK // 16)],
                                      e_v, sem)
                i1.wait()
                i2.wait()
                i3.wait()

                def off_body(r, c2):
                    for k in range(8):
                        sl = pl.ds(k * 16, 16)
                        dsto_v[r, sl] = dst_v[r, sl] + qoff
                        srco_v[r, sl] = srco_v[r, sl] + qoff
                    return c2

                lax.fori_loop(0, 8, off_body, 0)

                copies = []
                for j in range(8):
                    copies.append(pltpu.async_copy(
                        tab_a.at[dsto_v.at[j]],
                        a_v.at[pl.ds(j * 128, 128)], sem))
                    copies.append(pltpu.async_copy(
                        tab_b.at[srco_v.at[j]],
                        b_v.at[pl.ds(j * 128, 128)], sem))
                for cp in copies:
                    cp.wait()

                fe16 = fe_v[...]

                @plsc.parallel_loop(0, CHUNK // 16)
                def _(g):
                    ev16 = e_v[g]
                    for l in range(16):
                        i = g * 16 + l
                        x2 = a_v[i] + b_v[i] + _lane_splat(ev16, l) * fe16
                        t_v[i] = 1.0 - 2.0 / (jnp.exp(x2) + 1.0)

                for j in range(8):
                    pltpu.sync_copy(t_v.at[pl.ds(j * 128, 128)],
                                    s_sh.at[dst_v.at[j]], add=True)

                if half == 0:
                    @pl.when(cid == 0)
                    def _():
                        for j in range(8):
                            pltpu.sync_copy(ones_v, deg_sh.at[dst_v.at[j]],
                                            add=True)

                return carry

            lax.fori_loop(0, NCHUNK, chunk_body, 0)
            plsc.subcore_barrier()

            pltpu.sync_copy(s_sh.at[pl.ds(r0, ROWS_PT)],
                            s_out.at[pl.ds(qoff + r0, ROWS_PT)])
            if half == 0:
                plsc.subcore_barrier()

        @pl.when(cid == 0)
        def _():
            pltpu.sync_copy(deg_sh.at[pl.ds(r0, ROWS_PT)],
                            deg_out.at[pl.ds(r0, ROWS_PT)])

    return sc_edge


_sc_edge = _make_sc_edge()


def _dot(a, b):
    return jnp.dot(a, b, preferred_element_type=jnp.float32)


def _quarters(out_ref, x):
    for qq in range(4):
        out_ref[qq] = x[:, qq * 16:(qq + 1) * 16]


def _tc1_body(cf_ref, vf_ref, cw1, cb1, cw2, cb2, vw1, vb1, vw2, vb2,
              a1w, a1b, b1w, a2w, a2b,
              c_out, v_out, a1_out, b1_out, a2_out, zs_out, zd_out):
    zs_out[...] = jnp.zeros_like(zs_out)
    zd_out[...] = jnp.zeros_like(zd_out)
    c = jnp.tanh(_dot(jnp.tanh(_dot(cf_ref[...], cw1[...]) + cb1[...]),
                      cw2[...]) + cb2[...])
    v = jnp.tanh(_dot(jnp.tanh(_dot(vf_ref[...], vw1[...]) + vb1[...]),
                      vw2[...]) + vb2[...])
    c_out[...] = c
    v_out[...] = v
    _quarters(a1_out, 2.0 * (_dot(c, a1w[...]) + a1b[...]))
    _quarters(b1_out, 2.0 * _dot(v, b1w[...]))
    _quarters(a2_out, 2.0 * (_dot(v, a2w[...]) + a2b[...]))


def _tc2_body(s_ref, deg_ref, c_ref, ffo1, fbo1, o1bot, o1b, o2w, o2b, frw,
              b2_out):
    s = jnp.concatenate([s_ref[0], s_ref[1], s_ref[2], s_ref[3]], axis=-1)
    z = (_dot(s, ffo1[...]) + deg_ref[...] * fbo1[...]
         + _dot(c_ref[...], o1bot[...]) + o1b[...])
    cnew = _dot(jnp.tanh(z), o2w[...]) + o2b[...]
    _quarters(b2_out, 2.0 * _dot(cnew, frw[...]))


def _tc3_body(s_ref, deg_ref, v_ref, ffo1, fbo1, o1bot, o1b, o2w, o2b,
              w1, b1, w2, out_ref):
    s = jnp.concatenate([s_ref[0], s_ref[1], s_ref[2], s_ref[3]], axis=-1)
    z = (_dot(s, ffo1[...]) + deg_ref[...] * fbo1[...]
         + _dot(v_ref[...], o1bot[...]) + o1b[...])
    vnew = _dot(jnp.tanh(z), o2w[...]) + o2b[...]
    out_ref[...] = _dot(jnp.tanh(_dot(vnew, w1[...]) + b1[...]), w2[...])


def _row_spec(width):
    return pl.BlockSpec((BLK, width), lambda i: (i, 0))


def _q_spec():
    return pl.BlockSpec((4, BLK, 16), lambda i: (0, i, 0))


def _w_spec(shape):
    nd = len(shape)
    return pl.BlockSpec(shape, lambda i, _n=nd: (0,) * _n)


def kernel(constraint_features, edge_indices, edge_features,
           variable_features, params):
    p = params
    f32 = jnp.float32

    cfp = jnp.pad(constraint_features, ((0, NP - N), (0, 3)))
    vfp = jnp.pad(variable_features, ((0, NP - N), (0, 7)))
    cw1 = jnp.pad(p["c_emb"]["W1"], ((0, 3), (0, 0)))
    vw1 = jnp.pad(p["v_emb"]["W1"], ((0, 7), (0, 0)))

    def row(x):
        return x.reshape(1, -1).astype(f32)

    pad_i = jnp.full((EP - E,), N, jnp.int32)
    d1 = jnp.concatenate([edge_indices[0], pad_i])
    s1 = jnp.concatenate([edge_indices[1], pad_i])
    dst1 = d1.reshape(EP // 128, 128)
    dst2 = s1.reshape(EP // 128, 128)
    e_pad = jnp.concatenate([edge_features[:, 0],
                             jnp.zeros((EP - E,), f32)]).reshape(EP // 16, 16)
    ones_h = jnp.ones((128, 1), f32)
    fe1 = 2.0 * p["vc"]["fe_W"].reshape(EMB)
    fe2 = 2.0 * p["cv"]["fe_W"].reshape(EMB)

    def conv_consts(q):
        o1t, o1bot = q["o1_W"][:EMB], q["o1_W"][EMB:]
        return (q["ff_W"] @ o1t, row(q["ff_b"] @ o1t), o1bot,
                row(q["o1_b"]), q["o2_W"], row(q["o2_b"]))

    ffo1_1, fbo1_1, o1bot_1, o1b_1, o2w_1, o2b_1 = conv_consts(p["vc"])
    ffo1_2, fbo1_2, o1bot_2, o1b_2, o2w_2, o2b_2 = conv_consts(p["cv"])

    grid = NP // BLK
    w = _w_spec
    c, v, a1, b1, a2, z_s, z_d = pl.pallas_call(
        _tc1_body,
        grid=(grid,),
        in_specs=[_row_spec(8), _row_spec(24),
                  w((8, EMB)), w((1, EMB)), w((EMB, EMB)), w((1, EMB)),
                  w((24, EMB)), w((1, EMB)), w((EMB, EMB)), w((1, EMB)),
                  w((EMB, EMB)), w((1, EMB)), w((EMB, EMB)),
                  w((EMB, EMB)), w((1, EMB))],
        out_specs=[_row_spec(EMB), _row_spec(EMB),
                   _q_spec(), _q_spec(), _q_spec(),
                   _row_spec(16), _row_spec(1)],
        out_shape=[jax.ShapeDtypeStruct((NP, EMB), f32),
                   jax.ShapeDtypeStruct((NP, EMB), f32),
                   jax.ShapeDtypeStruct((4, NP, 16), f32),
                   jax.ShapeDtypeStruct((4, NP, 16), f32),
                   jax.ShapeDtypeStruct((4, NP, 16), f32),
                   jax.ShapeDtypeStruct((NP, 16), f32),
                   jax.ShapeDtypeStruct((NP, 1), f32)],
    )(cfp, vfp,
      cw1, row(p["c_emb"]["b1"]), p["c_emb"]["W2"], row(p["c_emb"]["b2"]),
      vw1, row(p["v_emb"]["b1"]), p["v_emb"]["W2"], row(p["v_emb"]["b2"]),
      p["vc"]["fl_W"], row(p["vc"]["fl_b"]), p["vc"]["fr_W"],
      p["cv"]["fl_W"], row(p["cv"]["fl_b"]))

    s1_out, deg1 = _sc_edge(dst1, dst2, e_pad,
                            a1.reshape(4 * NP, 16), b1.reshape(4 * NP, 16),
                            fe1, z_s, z_d, ones_h)

    (b2,) = pl.pallas_call(
        _tc2_body,
        grid=(grid,),
        in_specs=[_q_spec(), _row_spec(1), _row_spec(EMB),
                  w((EMB, EMB)), w((1, EMB)), w((EMB, EMB)), w((1, EMB)),
                  w((EMB, EMB)), w((1, EMB)), w((EMB, EMB))],
        out_specs=[_q_spec()],
        out_shape=[jax.ShapeDtypeStruct((4, NP, 16), f32)],
    )(s1_out.reshape(4, NP, 16), deg1, c,
      ffo1_1, fbo1_1, o1bot_1, o1b_1, o2w_1, o2b_1, p["cv"]["fr_W"])

    s2_out, deg2 = _sc_edge(dst2, dst1, e_pad,
                            a2.reshape(4 * NP, 16), b2.reshape(4 * NP, 16),
                            fe2, z_s, z_d, ones_h)

    (out,) = pl.pallas_call(
        _tc3_body,
        grid=(grid,),
        in_specs=[_q_spec(), _row_spec(1), _row_spec(EMB),
                  w((EMB, EMB)), w((1, EMB)), w((EMB, EMB)), w((1, EMB)),
                  w((EMB, EMB)), w((1, EMB)),
                  w((EMB, EMB)), w((1, EMB)), w((EMB, 1))],
        out_specs=[_row_spec(1)],
        out_shape=[jax.ShapeDtypeStruct((NP, 1), f32)],
    )(s2_out.reshape(4, NP, 16), deg2, v,
      ffo1_2, fbo1_2, o1bot_2, o1b_2, o2w_2, o2b_2,
      p["out"]["W1"], row(p["out"]["b1"]), p["out"]["W2"])

    return out[:N]
